# baseline (device time: 24141 ns/iter reference)
import jax
import jax.numpy as jnp
from jax import lax
from jax.experimental import pallas as pl
from jax.experimental.pallas import tpu as pltpu

Z = 4
BLK = 256


def kernel(x, dy, gamma):
    m, d = x.shape
    grid = m // BLK

    def body(x_ref, dy_ref, gamma_ref, out_ref, acc_ref, comm_ref,
             send_sems, recv_sems):
        step = pl.program_id(0)

        xv = x_ref[:, :]
        dyv = dy_ref[:, :]
        s1 = jnp.sum(xv, axis=1, keepdims=True)
        s2 = jnp.sum(xv * xv, axis=1, keepdims=True)
        mu = s1 * (1.0 / d)
        var = s2 * (1.0 / d) - mu * mu
        rstd = lax.rsqrt(var + 1e-5)
        b = rstd * mu
        dgamma = jnp.sum(rstd * (dyv * xv) - b * dyv, axis=0)[None, :]
        dbeta = jnp.sum(dyv, axis=0)[None, :]
        part = jnp.concatenate([dgamma, dbeta], axis=0)

        @pl.when(step == 0)
        def _():
            acc_ref[:, :] = part

        @pl.when(step != 0)
        def _():
            acc_ref[:, :] = acc_ref[:, :] + part

        @pl.when(step == grid - 1)
        def _():
            my_x = lax.axis_index("x")
            my_y = lax.axis_index("y")
            my_z = lax.axis_index("z")

            comm_ref[pl.ds(my_z, 1)] = acc_ref[:, :][None]

            barrier_sem = pltpu.get_barrier_semaphore()
            for off in range(1, Z):
                pl.semaphore_signal(
                    barrier_sem,
                    inc=1,
                    device_id=(my_x, my_y, lax.rem(my_z + off, Z)),
                    device_id_type=pl.DeviceIdType.MESH,
                )
            pl.semaphore_wait(barrier_sem, Z - 1)

            sends = []
            for off in range(1, Z):
                rdma = pltpu.make_async_remote_copy(
                    src_ref=comm_ref.at[my_z],
                    dst_ref=comm_ref.at[my_z],
                    send_sem=send_sems.at[off - 1],
                    recv_sem=recv_sems.at[my_z],
                    device_id=(my_x, my_y, lax.rem(my_z + off, Z)),
                    device_id_type=pl.DeviceIdType.MESH,
                )
                rdma.start()
                sends.append(rdma)

            for off in range(1, Z):
                src_z = lax.rem(my_z + Z - off, Z)
                recv = pltpu.make_async_remote_copy(
                    src_ref=comm_ref.at[src_z],
                    dst_ref=comm_ref.at[src_z],
                    send_sem=send_sems.at[off - 1],
                    recv_sem=recv_sems.at[src_z],
                    device_id=(my_x, my_y, my_z),
                    device_id_type=pl.DeviceIdType.MESH,
                )
                recv.wait_recv()
            for s in sends:
                s.wait_send()

            out_ref[:, :] = (
                comm_ref[0] + comm_ref[1] + comm_ref[2] + comm_ref[3]
            )

    return pl.pallas_call(
        body,
        grid=(grid,),
        out_shape=jax.ShapeDtypeStruct((2, d), jnp.float32),
        in_specs=[
            pl.BlockSpec((BLK, d), lambda i: (i, 0)),
            pl.BlockSpec((BLK, d), lambda i: (i, 0)),
            pl.BlockSpec(memory_space=pl.ANY),
        ],
        out_specs=pl.BlockSpec((2, d), lambda i: (0, 0)),
        scratch_shapes=[
            pltpu.VMEM((2, d), jnp.float32),
            pltpu.VMEM((Z, 2, d), jnp.float32),
            pltpu.SemaphoreType.DMA((Z - 1,)),
            pltpu.SemaphoreType.DMA((Z,)),
        ],
        compiler_params=pltpu.CompilerParams(collective_id=0),
    )(x, dy, gamma)


# device time: 21089 ns/iter; 1.1447x vs baseline; 1.1447x over previous
import jax
import jax.numpy as jnp
from jax import lax
from jax.experimental import pallas as pl
from jax.experimental.pallas import tpu as pltpu

Z = 4
BLK = 512


def kernel(x, dy, gamma):
    m, d = x.shape
    grid = m // BLK

    def body(x_ref, dy_ref, gamma_ref, out_ref, acc_ref, comm_ref,
             send_sems, recv_sems):
        step = pl.program_id(0)
        my_x = lax.axis_index("x")
        my_y = lax.axis_index("y")
        my_z = lax.axis_index("z")
        barrier_sem = pltpu.get_barrier_semaphore()

        @pl.when(step == 0)
        def _():
            for off in range(1, Z):
                pl.semaphore_signal(
                    barrier_sem,
                    inc=1,
                    device_id=(my_x, my_y, lax.rem(my_z + off, Z)),
                    device_id_type=pl.DeviceIdType.MESH,
                )

        xv = x_ref[:, :]
        dyv = dy_ref[:, :]
        s1 = jnp.sum(xv, axis=1, keepdims=True)
        s2 = jnp.sum(xv * xv, axis=1, keepdims=True)
        mu = s1 * (1.0 / d)
        var = s2 * (1.0 / d) - mu * mu
        rstd = lax.rsqrt(var + 1e-5)
        b = rstd * mu
        dgamma = jnp.sum(dyv * (rstd * xv - b), axis=0)[None, :]
        dbeta = jnp.sum(dyv, axis=0)[None, :]
        part = jnp.concatenate([dgamma, dbeta], axis=0)

        @pl.when(step == 0)
        def _():
            acc_ref[:, :] = part

        @pl.when(step != 0)
        def _():
            acc_ref[:, :] = acc_ref[:, :] + part

        @pl.when(step == grid - 1)
        def _():
            comm_ref[pl.ds(my_z, 1)] = acc_ref[:, :][None]
            pl.semaphore_wait(barrier_sem, Z - 1)

            sends = []
            for off in range(1, Z):
                rdma = pltpu.make_async_remote_copy(
                    src_ref=comm_ref.at[my_z],
                    dst_ref=comm_ref.at[my_z],
                    send_sem=send_sems.at[off - 1],
                    recv_sem=recv_sems.at[my_z],
                    device_id=(my_x, my_y, lax.rem(my_z + off, Z)),
                    device_id_type=pl.DeviceIdType.MESH,
                )
                rdma.start()
                sends.append(rdma)

            for off in range(1, Z):
                src_z = lax.rem(my_z + Z - off, Z)
                recv = pltpu.make_async_remote_copy(
                    src_ref=comm_ref.at[src_z],
                    dst_ref=comm_ref.at[src_z],
                    send_sem=send_sems.at[off - 1],
                    recv_sem=recv_sems.at[src_z],
                    device_id=(my_x, my_y, my_z),
                    device_id_type=pl.DeviceIdType.MESH,
                )
                recv.wait_recv()
            for s in sends:
                s.wait_send()

            out_ref[:, :] = (
                comm_ref[0] + comm_ref[1] + comm_ref[2] + comm_ref[3]
            )

    return pl.pallas_call(
        body,
        grid=(grid,),
        out_shape=jax.ShapeDtypeStruct((2, d), jnp.float32),
        in_specs=[
            pl.BlockSpec((BLK, d), lambda i: (i, 0)),
            pl.BlockSpec((BLK, d), lambda i: (i, 0)),
            pl.BlockSpec(memory_space=pl.ANY),
        ],
        out_specs=pl.BlockSpec((2, d), lambda i: (0, 0)),
        scratch_shapes=[
            pltpu.VMEM((2, d), jnp.float32),
            pltpu.VMEM((Z, 2, d), jnp.float32),
            pltpu.SemaphoreType.DMA((Z - 1,)),
            pltpu.SemaphoreType.DMA((Z,)),
        ],
        compiler_params=pltpu.CompilerParams(collective_id=0),
    )(x, dy, gamma)


# device time: 20342 ns/iter; 1.1868x vs baseline; 1.0367x over previous
import jax
import jax.numpy as jnp
from jax import lax
from jax.experimental import pallas as pl
from jax.experimental.pallas import tpu as pltpu

Z = 4
BLK = 512


def kernel(x, dy, gamma):
    m, d = x.shape
    grid = m // BLK

    def body(x_ref, dy_ref, out_ref, acc_ref, comm_ref, vmem_filler_ref,
             send_sems, recv_sems):
        step = pl.program_id(0)
        my_x = lax.axis_index("x")
        my_y = lax.axis_index("y")
        my_z = lax.axis_index("z")
        barrier_sem = pltpu.get_barrier_semaphore()

        @pl.when(step == 0)
        def _():
            for off in range(1, Z):
                pl.semaphore_signal(
                    barrier_sem,
                    inc=1,
                    device_id=(my_x, my_y, lax.rem(my_z + off, Z)),
                    device_id_type=pl.DeviceIdType.MESH,
                )

        xv = x_ref[:, :]
        dyv = dy_ref[:, :]
        s1 = jnp.sum(xv, axis=1, keepdims=True)
        s2 = jnp.sum(xv * xv, axis=1, keepdims=True)
        mu = s1 * (1.0 / d)
        var = s2 * (1.0 / d) - mu * mu
        rstd = lax.rsqrt(var + 1e-5)
        b = rstd * mu
        dgamma = jnp.sum(dyv * (rstd * xv - b), axis=0)[None, :]
        dbeta = jnp.sum(dyv, axis=0)[None, :]
        part = jnp.concatenate([dgamma, dbeta], axis=0)

        @pl.when(step == 0)
        def _():
            acc_ref[:, :] = part

        @pl.when(step != 0)
        def _():
            acc_ref[:, :] = acc_ref[:, :] + part

        @pl.when(step == grid - 1)
        def _():
            comm_ref[pl.ds(my_z, 1)] = acc_ref[:, :][None]
            pl.semaphore_wait(barrier_sem, Z - 1)

            sends = []
            for off in range(1, Z):
                rdma = pltpu.make_async_remote_copy(
                    src_ref=comm_ref.at[my_z],
                    dst_ref=comm_ref.at[my_z],
                    send_sem=send_sems.at[off - 1],
                    recv_sem=recv_sems.at[my_z],
                    device_id=(my_x, my_y, lax.rem(my_z + off, Z)),
                    device_id_type=pl.DeviceIdType.MESH,
                )
                rdma.start()
                sends.append(rdma)

            for off in range(1, Z):
                src_z = lax.rem(my_z + Z - off, Z)
                recv = pltpu.make_async_remote_copy(
                    src_ref=comm_ref.at[src_z],
                    dst_ref=comm_ref.at[src_z],
                    send_sem=send_sems.at[off - 1],
                    recv_sem=recv_sems.at[src_z],
                    device_id=(my_x, my_y, my_z),
                    device_id_type=pl.DeviceIdType.MESH,
                )
                recv.wait_recv()
            for s in sends:
                s.wait_send()

            out_ref[:, :] = (
                comm_ref[0] + comm_ref[1] + comm_ref[2] + comm_ref[3]
            )

    return pl.pallas_call(
        body,
        grid=(grid,),
        out_shape=jax.ShapeDtypeStruct((2, d), jnp.float32),
        in_specs=[
            pl.BlockSpec((BLK, d), lambda i: (i, 0)),
            pl.BlockSpec((BLK, d), lambda i: (i, 0)),
        ],
        out_specs=pl.BlockSpec((2, d), lambda i: (0, 0)),
        scratch_shapes=[
            pltpu.VMEM((2, d), jnp.float32),
            pltpu.VMEM((Z, 2, d), jnp.float32),
            pltpu.VMEM((10 * 1024 * 1024,), jnp.float32),
            pltpu.SemaphoreType.DMA((Z - 1,)),
            pltpu.SemaphoreType.DMA((Z,)),
        ],
        compiler_params=pltpu.CompilerParams(collective_id=0),
    )(x, dy)
